# baseline (speedup 1.0000x reference)
import functools

import jax
import jax.numpy as jnp
from jax import lax
from jax.experimental import pallas as pl
from jax.experimental.pallas import tpu as pltpu
from jax.experimental.pallas import tpu_sc as plsc

N_FACTORS = 16
LANES = 16
CHUNK = 128


@functools.lru_cache(maxsize=None)
def _build(B, nw):
    b_per_w = B // nw
    rows_per_w = b_per_w // CHUNK
    n_groups = b_per_w // LANES

    mesh = plsc.VectorSubcoreMesh(core_axis_name="c", subcore_axis_name="s")

    @functools.partial(
        pl.kernel,
        mesh=mesh,
        out_type=jax.ShapeDtypeStruct((B,), jnp.float32),
        compiler_params=pltpu.CompilerParams(
            needs_layout_passes=False, use_tc_tiling_on_sc=False),
        scratch_types=[
            pltpu.VMEM((rows_per_w, CHUNK), jnp.int32),
            pltpu.VMEM((rows_per_w, CHUNK), jnp.int32),
            pltpu.VMEM((b_per_w, N_FACTORS), jnp.float32),
            pltpu.VMEM((b_per_w, N_FACTORS), jnp.float32),
            pltpu.VMEM((b_per_w,), jnp.float32),
            pltpu.VMEM((b_per_w,), jnp.float32),
            pltpu.VMEM((LANES,), jnp.float32),
            pltpu.VMEM((b_per_w,), jnp.float32),
            pltpu.SemaphoreType.DMA,
        ],
    )
    def k(u2d, i2d, mu_hbm, bu_hbm, bi_hbm, p_hbm, q_hbm, out_hbm,
          ui, ii, pu, qi, bu, bi, muv, outv, sem):
        wid = lax.axis_index("s") * 2 + lax.axis_index("c")
        r0 = wid * rows_per_w
        pltpu.sync_copy(u2d.at[pl.ds(r0, rows_per_w)], ui)
        pltpu.sync_copy(i2d.at[pl.ds(r0, rows_per_w)], ii)
        pltpu.sync_copy(mu_hbm, muv)
        copies = []
        for j in range(rows_per_w):
            copies.append(pltpu.async_copy(
                p_hbm.at[ui.at[j]], pu.at[pl.ds(j * CHUNK, CHUNK)], sem))
            copies.append(pltpu.async_copy(
                q_hbm.at[ii.at[j]], qi.at[pl.ds(j * CHUNK, CHUNK)], sem))
            copies.append(pltpu.async_copy(
                bu_hbm.at[ui.at[j]], bu.at[pl.ds(j * CHUNK, CHUNK)], sem))
            copies.append(pltpu.async_copy(
                bi_hbm.at[ii.at[j]], bi.at[pl.ds(j * CHUNK, CHUNK)], sem))
        for c in copies:
            c.wait()
        mu16 = muv[...]
        lane = lax.iota(jnp.int32, LANES)
        for g in range(n_groups):
            rows = lane + g * LANES
            acc = bu[pl.ds(g * LANES, LANES)] + bi[pl.ds(g * LANES, LANES)] + mu16
            for kk in range(N_FACTORS):
                col = jnp.full((LANES,), kk, jnp.int32)
                p = plsc.load_gather(pu, [rows, col])
                q = plsc.load_gather(qi, [rows, col])
                acc = acc + p * q
            outv[pl.ds(g * LANES, LANES)] = acc
        pltpu.sync_copy(outv, out_hbm.at[pl.ds(wid * b_per_w, b_per_w)])

    return k


def kernel(u_idx, i_idx, mu, b_u, b_i, P, Q):
    B = u_idx.shape[0]
    info = plsc.get_sparse_core_info()
    nw = info.num_cores * info.num_subcores
    u2d = u_idx.astype(jnp.int32).reshape(B // CHUNK, CHUNK)
    i2d = i_idx.astype(jnp.int32).reshape(B // CHUNK, CHUNK)
    muv = jnp.full((LANES,), mu, jnp.float32)
    return _build(B, nw)(u2d, i2d, muv, b_u, b_i, P, Q)


# speedup vs baseline: 7.0479x; 7.0479x over previous
import functools

import jax
import jax.numpy as jnp
from jax import lax
from jax.experimental import pallas as pl
from jax.experimental.pallas import tpu as pltpu
from jax.experimental.pallas import tpu_sc as plsc

N_FACTORS = 16
LANES = 16
CHUNK = 128
ROW = 128


@functools.lru_cache(maxsize=None)
def _build(B, nw):
    b_per_w = B // nw
    n_chunks = b_per_w // CHUNK
    gr_per_chunk = CHUNK // LANES

    mesh = plsc.VectorSubcoreMesh(core_axis_name="c", subcore_axis_name="s")

    @functools.partial(
        pl.kernel,
        mesh=mesh,
        out_type=jax.ShapeDtypeStruct((B,), jnp.float32),
        compiler_params=pltpu.CompilerParams(
            needs_layout_passes=False, use_tc_tiling_on_sc=False),
        scratch_types=[
            pltpu.VMEM((b_per_w,), jnp.int32),
            pltpu.VMEM((b_per_w,), jnp.int32),
            pltpu.VMEM((n_chunks, CHUNK), jnp.int32),
            pltpu.VMEM((n_chunks, CHUNK), jnp.int32),
            pltpu.VMEM((2, CHUNK, ROW), jnp.float32),
            pltpu.VMEM((2, CHUNK, ROW), jnp.float32),
            pltpu.VMEM((b_per_w,), jnp.float32),
            pltpu.VMEM((b_per_w,), jnp.float32),
            pltpu.VMEM((LANES,), jnp.float32),
            pltpu.VMEM((b_per_w,), jnp.float32),
            pltpu.SemaphoreType.DMA,
            pltpu.SemaphoreType.DMA,
        ],
    )
    def k(u_hbm, i_hbm, mu_hbm, bu_hbm, bi_hbm, pp_hbm, qp_hbm, out_hbm,
          ui, ii, rp, rq, pub, qib, bu, bi, muv, outv, sem_pq, sem_b):
        wid = lax.axis_index("s") * 2 + lax.axis_index("c")
        base = wid * b_per_w
        pltpu.sync_copy(u_hbm.at[pl.ds(base, b_per_w)], ui)
        pltpu.sync_copy(i_hbm.at[pl.ds(base, b_per_w)], ii)
        pltpu.sync_copy(mu_hbm, muv)
        bias_copies = [
            pltpu.async_copy(bu_hbm.at[ui], bu, sem_b),
            pltpu.async_copy(bi_hbm.at[ii], bi, sem_b),
        ]
        for j in range(n_chunks):
            for t in range(gr_per_chunk):
                o = j * CHUNK + t * LANES
                uv = ui[pl.ds(o, LANES)]
                iv = ii[pl.ds(o, LANES)]
                rp[j, pl.ds(t * LANES, LANES)] = lax.shift_left(
                    lax.shift_right_logical(uv, 10), 7) + jnp.bitwise_and(uv, 127)
                rq[j, pl.ds(t * LANES, LANES)] = lax.shift_left(
                    lax.shift_right_logical(iv, 10), 7) + jnp.bitwise_and(iv, 127)

        def fire(j):
            return (
                pltpu.async_copy(pp_hbm.at[rp.at[j]], pub.at[j % 2], sem_pq),
                pltpu.async_copy(qp_hbm.at[rq.at[j]], qib.at[j % 2], sem_pq),
            )

        for c in bias_copies:
            c.wait()
        mu16 = muv[...]
        lane = lax.iota(jnp.int32, LANES)
        pending = fire(0)
        for j in range(n_chunks):
            for c in pending:
                c.wait()
            if j + 1 < n_chunks:
                pending = fire(j + 1)
            for t in range(gr_per_chunk):
                o = j * CHUNK + t * LANES
                uv = ui[pl.ds(o, LANES)]
                iv = ii[pl.ds(o, LANES)]
                cu = lax.shift_left(
                    jnp.bitwise_and(lax.shift_right_logical(uv, 7), 7), 4)
                ci = lax.shift_left(
                    jnp.bitwise_and(lax.shift_right_logical(iv, 7), 7), 4)
                pos = lane + t * LANES
                acc = bu[pl.ds(o, LANES)] + bi[pl.ds(o, LANES)] + mu16
                for kk in range(N_FACTORS):
                    p = plsc.load_gather(pub.at[j % 2], [pos, cu + kk])
                    q = plsc.load_gather(qib.at[j % 2], [pos, ci + kk])
                    acc = acc + p * q
                outv[pl.ds(o, LANES)] = acc
        pltpu.sync_copy(outv, out_hbm.at[pl.ds(base, b_per_w)])

    return k


_TC_COLS = 98304
_TC_OUT = _TC_COLS // 8
_TC_SUB = _TC_COLS // 1024


@functools.lru_cache(maxsize=None)
def _tc_repack(n_rows):
    grid = -(-n_rows // _TC_COLS)
    out_sds = jax.ShapeDtypeStruct((grid * _TC_OUT, ROW), jnp.float32)

    def body(pt_ref, qt_ref, pp_ref, qp_ref):
        for src, dst in ((pt_ref, pp_ref), (qt_ref, qp_ref)):
            for t in range(_TC_SUB):
                x = src[:, pl.ds(t * 1024, 1024)]
                x2 = (x.reshape(N_FACTORS, 8, ROW)
                      .swapaxes(0, 1)
                      .reshape(ROW, ROW))
                dst[pl.ds(t * ROW, ROW), :] = x2.T

    spec_in = pl.BlockSpec((N_FACTORS, _TC_COLS), lambda c: (0, c))
    spec_out = pl.BlockSpec((_TC_OUT, ROW), lambda c: (c, 0))
    return pl.pallas_call(
        body,
        grid=(grid,),
        in_specs=[spec_in, spec_in],
        out_specs=[spec_out, spec_out],
        out_shape=[out_sds, out_sds],
        compiler_params=pltpu.CompilerParams(
            dimension_semantics=("arbitrary",)),
    )


def kernel(u_idx, i_idx, mu, b_u, b_i, P, Q):
    B = u_idx.shape[0]
    info = plsc.get_sparse_core_info()
    nw = info.num_cores * info.num_subcores
    muv = jnp.full((LANES,), mu, jnp.float32)
    pp, qp = _tc_repack(P.shape[0])(P.T, Q.T)
    return _build(B, nw)(
        u_idx.astype(jnp.int32), i_idx.astype(jnp.int32), muv,
        b_u, b_i, pp, qp)
